# baseline (speedup 1.0000x reference)
import functools

import jax
import jax.numpy as jnp
from jax import lax
from jax.experimental import pallas as pl
from jax.experimental.pallas import tpu as pltpu
from jax.experimental.pallas import tpu_sc as plsc

_THRESHOLD = 0.1

_B = 4
_S = 4096
_D = 4096
_HALF_ROWS = _S // 2
_NW = 32
_BAND = _HALF_ROWS // _NW
_CR = 8
_CC = 2048
_RC = _BAND // _CR
_CCN = _D // _CC
_PER_HALF = _B * _RC * _CCN
_NCH = 2 * _PER_HALF
_NB = 4
_PREF = 2


def _mask_chunk(buf):
    for r in range(_CR):
        @plsc.parallel_loop(0, _CC, 16, unroll=8)
        def _m(i):
            v = buf[r, pl.ds(i, 16)]
            buf[r, pl.ds(i, 16)] = jnp.where(jnp.abs(v) > _THRESHOLD, v, 0.0)


def _sc_body(x_hbm, o_hbm, *scratch):
    bufs = scratch[:_NB]
    isems = scratch[_NB:2 * _NB]
    osems = scratch[2 * _NB:3 * _NB]
    wid = lax.axis_index("s") * 2 + lax.axis_index("c")

    def addr(h):
        c = h // 2
        m = h % 2
        b = c // (_RC * _CCN)
        rc = (c // _CCN) % _RC
        cc = c % _CCN
        row = m * _HALF_ROWS + wid * _BAND + rc * _CR
        return b, pl.multiple_of(row, _CR), cc * _CC

    def in_dma(h, slot):
        b, row, col = addr(h)
        return pltpu.make_async_copy(
            x_hbm.at[b, pl.ds(row, _CR), pl.ds(col, _CC)],
            bufs[slot],
            isems[slot],
        )

    def out_dma(h, slot):
        b, row, col = addr(h)
        return pltpu.make_async_copy(
            bufs[slot],
            o_hbm.at[b, pl.ds(row, _CR), pl.ds(col, _CC)],
            osems[slot],
        )

    for s in range(_PREF):
        in_dma(s, s).start()

    def step(k, _):
        for s in range(_NB):
            h = k * _NB + s
            if s >= _PREF:
                out_dma(h - _PREF, (s - _PREF) % _NB).wait()
            else:
                @pl.when(k > 0)
                def _w():
                    out_dma(h - _PREF, (s - _PREF) % _NB).wait()

            @pl.when(h + _PREF < _NCH)
            def _p():
                in_dma(h + _PREF, (s + _PREF) % _NB).start()

            in_dma(h, s).wait()
            if s % 2 == 1:
                _mask_chunk(bufs[s])
            out_dma(h, s).start()
        return _

    lax.fori_loop(0, _NCH // _NB, step, 0)

    for h in range(_NCH - _PREF, _NCH):
        out_dma(h, h % _NB).wait()


_sc_kernel = functools.partial(
    pl.kernel,
    out_type=jax.ShapeDtypeStruct((_B, _S, _D), jnp.float32),
    mesh=plsc.VectorSubcoreMesh(core_axis_name="c", subcore_axis_name="s"),
    scratch_types=(
        [pltpu.VMEM((_CR, _CC), jnp.float32)] * _NB
        + [pltpu.SemaphoreType.DMA] * (2 * _NB)
    ),
    compiler_params=pltpu.CompilerParams(use_tc_tiling_on_sc=True),
)(_sc_body)


def kernel(x):
    return _sc_kernel(x)


# speedup vs baseline: 1.0251x; 1.0251x over previous
import functools

import jax
import jax.numpy as jnp
from jax import lax
from jax.experimental import pallas as pl
from jax.experimental.pallas import tpu as pltpu
from jax.experimental.pallas import tpu_sc as plsc

_THRESHOLD = 0.1

_B = 4
_S = 4096
_D = 4096
_HALF_ROWS = _S // 2
_NW = 32
_NS = 16
_BAND = _HALF_ROWS // _NW
_CR = 8
_CC = 2048
_RC = _BAND // _CR
_CCN = _D // _CC
_NCH = _B * _RC * _CCN
_NB = 4
_PREF = 2


def _mask_chunk(buf):
    for r in range(_CR):
        @plsc.parallel_loop(0, _CC, 16, unroll=8)
        def _m(i):
            v = buf[r, pl.ds(i, 16)]
            buf[r, pl.ds(i, 16)] = jnp.where(jnp.abs(v) > _THRESHOLD, v, 0.0)


def _sc_body(x_hbm, o_hbm, spm, *scratch):
    bufs = scratch[:_NB]
    isems = scratch[_NB:2 * _NB]
    osems = scratch[2 * _NB:3 * _NB]
    cisems = scratch[3 * _NB:4 * _NB]
    cosems = scratch[4 * _NB:5 * _NB]
    sid = lax.axis_index("s")
    wid = sid * 2 + lax.axis_index("c")

    def addr(c, masked):
        b = c // (_RC * _CCN)
        rc = (c // _CCN) % _RC
        cc = c % _CCN
        row = masked * _HALF_ROWS + wid * _BAND + rc * _CR
        return b, pl.multiple_of(row, _CR), cc * _CC

    def hbm_ref(ref, c, masked):
        b, row, col = addr(c, masked)
        return ref.at[b, pl.ds(row, _CR), pl.ds(col, _CC)]

    def m_in(c, slot):
        return pltpu.make_async_copy(hbm_ref(x_hbm, c, 1), bufs[slot], isems[slot])

    def m_out(c, slot):
        return pltpu.make_async_copy(bufs[slot], hbm_ref(o_hbm, c, 1), osems[slot])

    def c_in(c, slot):
        return pltpu.make_async_copy(
            hbm_ref(x_hbm, c, 0), spm.at[sid, slot], cisems[slot]
        )

    def c_out(c, slot):
        return pltpu.make_async_copy(
            spm.at[sid, slot], hbm_ref(o_hbm, c, 0), cosems[slot]
        )

    def process(h, s, drain, prefetch):
        slot_p = (s + _PREF) % _NB
        if drain:
            m_out(h - (_NB - _PREF), slot_p).wait()
            c_out(h - (_NB - _PREF), slot_p).wait()
        if prefetch:
            m_in(h + _PREF, slot_p).start()
            c_in(h + _PREF, slot_p).start()
        c_in(h, s).wait()
        c_out(h, s).start()
        m_in(h, s).wait()
        _mask_chunk(bufs[s])
        m_out(h, s).start()

    for s in range(_PREF):
        m_in(s, s).start()
        c_in(s, s).start()

    for s in range(_NB):
        process(s, s, drain=(s >= _NB - _PREF), prefetch=True)

    def step(k, _):
        for s in range(_NB):
            process(k * _NB + s, s, drain=True, prefetch=True)
        return _

    lax.fori_loop(1, _NCH // _NB - 1, step, 0)

    last = _NCH - _NB
    for s in range(_NB):
        h = last + s
        process(h, s, drain=(h + _PREF < _NCH), prefetch=(h + _PREF < _NCH))

    for s in range(_NB):
        h = last + s
        m_out(h, s).wait()
        c_out(h, s).wait()


_sc_kernel = functools.partial(
    pl.kernel,
    out_type=jax.ShapeDtypeStruct((_B, _S, _D), jnp.float32),
    mesh=plsc.VectorSubcoreMesh(core_axis_name="c", subcore_axis_name="s"),
    scratch_types=(
        [pltpu.VMEM_SHARED((_NS, _NB, _CR, _CC), jnp.float32)]
        + [pltpu.VMEM((_CR, _CC), jnp.float32)] * _NB
        + [pltpu.SemaphoreType.DMA] * (4 * _NB)
    ),
    compiler_params=pltpu.CompilerParams(use_tc_tiling_on_sc=True),
)(_sc_body)


def kernel(x):
    return _sc_kernel(x)


# speedup vs baseline: 1.0522x; 1.0265x over previous
import functools

import jax
import jax.numpy as jnp
from jax import lax
from jax.experimental import pallas as pl
from jax.experimental.pallas import tpu as pltpu
from jax.experimental.pallas import tpu_sc as plsc

_THRESHOLD = 0.1

_B = 4
_S = 4096
_D = 4096
_HALF_ROWS = _S // 2
_NW = 32
_NS = 16
_BAND = _HALF_ROWS // _NW
_CR = 8
_CC = 1024
_RC = _BAND // _CR
_CCN = _D // _CC
_NCH = _B * _RC * _CCN
_NB = 8
_PREF = 5


def _mask_chunk(buf):
    @plsc.parallel_loop(0, _CR * _CC, 16, unroll=8)
    def _m(i):
        r = i // _CC
        c = pl.multiple_of(i % _CC, 16)
        v = buf[r, pl.ds(c, 16)]
        buf[r, pl.ds(c, 16)] = jnp.where(jnp.abs(v) > _THRESHOLD, v, 0.0)


def _sc_body(x_hbm, o_hbm, spm, *scratch):
    bufs = scratch[:_NB]
    isems = scratch[_NB:2 * _NB]
    osems = scratch[2 * _NB:3 * _NB]
    cisems = scratch[3 * _NB:4 * _NB]
    cosems = scratch[4 * _NB:5 * _NB]
    sid = lax.axis_index("s")
    wid = sid * 2 + lax.axis_index("c")

    def addr(c, masked):
        b = c // (_RC * _CCN)
        rc = (c // _CCN) % _RC
        cc = c % _CCN
        row = masked * _HALF_ROWS + wid * _BAND + rc * _CR
        return b, pl.multiple_of(row, _CR), cc * _CC

    def hbm_ref(ref, c, masked):
        b, row, col = addr(c, masked)
        return ref.at[b, pl.ds(row, _CR), pl.ds(col, _CC)]

    def m_in(c, slot):
        return pltpu.make_async_copy(hbm_ref(x_hbm, c, 1), bufs[slot], isems[slot])

    def m_out(c, slot):
        return pltpu.make_async_copy(bufs[slot], hbm_ref(o_hbm, c, 1), osems[slot])

    def c_in(c, slot):
        return pltpu.make_async_copy(
            hbm_ref(x_hbm, c, 0), spm.at[sid, slot], cisems[slot]
        )

    def c_out(c, slot):
        return pltpu.make_async_copy(
            spm.at[sid, slot], hbm_ref(o_hbm, c, 0), cosems[slot]
        )

    def process(h, s, drain, prefetch):
        slot_p = (s + _PREF) % _NB
        if drain:
            m_out(h - (_NB - _PREF), slot_p).wait()
            c_out(h - (_NB - _PREF), slot_p).wait()
        if prefetch:
            m_in(h + _PREF, slot_p).start()
            c_in(h + _PREF, slot_p).start()
        c_in(h, s).wait()
        c_out(h, s).start()
        m_in(h, s).wait()
        _mask_chunk(bufs[s])
        m_out(h, s).start()

    for s in range(_PREF):
        m_in(s, s).start()
        c_in(s, s).start()

    for s in range(_NB):
        process(s, s, drain=(s >= _NB - _PREF), prefetch=True)

    def step(k, _):
        for s in range(_NB):
            process(k * _NB + s, s, drain=True, prefetch=True)
        return _

    lax.fori_loop(1, _NCH // _NB - 1, step, 0)

    last = _NCH - _NB
    for s in range(_NB):
        h = last + s
        process(h, s, drain=(h + _PREF < _NCH), prefetch=(h + _PREF < _NCH))

    for s in range(_NB):
        h = last + s
        m_out(h, s).wait()
        c_out(h, s).wait()


_sc_kernel = functools.partial(
    pl.kernel,
    out_type=jax.ShapeDtypeStruct((_B, _S, _D), jnp.float32),
    mesh=plsc.VectorSubcoreMesh(core_axis_name="c", subcore_axis_name="s"),
    scratch_types=(
        [pltpu.VMEM_SHARED((_NS, _NB, _CR, _CC), jnp.float32)]
        + [pltpu.VMEM((_CR, _CC), jnp.float32)] * _NB
        + [pltpu.SemaphoreType.DMA] * (4 * _NB)
    ),
    compiler_params=pltpu.CompilerParams(use_tc_tiling_on_sc=True),
)(_sc_body)


def kernel(x):
    return _sc_kernel(x)


# speedup vs baseline: 1.0523x; 1.0001x over previous
import functools

import jax
import jax.numpy as jnp
from jax import lax
from jax.experimental import pallas as pl
from jax.experimental.pallas import tpu as pltpu
from jax.experimental.pallas import tpu_sc as plsc

_THRESHOLD = 0.1

_B = 4
_S = 4096
_D = 4096
_HALF_ROWS = _S // 2
_NW = 32
_NS = 16
_BAND = _HALF_ROWS // _NW
_CR = 8
_CC = 1024
_RC = _BAND // _CR
_CCN = _D // _CC
_NCH = _B * _RC * _CCN
_NB = 8
_PREF = 6


def _mask_chunk(buf):
    @plsc.parallel_loop(0, _CR * _CC, 16, unroll=8)
    def _m(i):
        r = i // _CC
        c = pl.multiple_of(i % _CC, 16)
        v = buf[r, pl.ds(c, 16)]
        buf[r, pl.ds(c, 16)] = jnp.where(jnp.abs(v) > _THRESHOLD, v, 0.0)


def _sc_body(x_hbm, o_hbm, spm, *scratch):
    bufs = scratch[:_NB]
    isems = scratch[_NB:2 * _NB]
    osems = scratch[2 * _NB:3 * _NB]
    cisems = scratch[3 * _NB:4 * _NB]
    cosems = scratch[4 * _NB:5 * _NB]
    sid = lax.axis_index("s")
    wid = sid * 2 + lax.axis_index("c")

    def addr(c, masked):
        b = c // (_RC * _CCN)
        rc = (c // _CCN) % _RC
        cc = c % _CCN
        row = masked * _HALF_ROWS + wid * _BAND + rc * _CR
        return b, pl.multiple_of(row, _CR), cc * _CC

    def hbm_ref(ref, c, masked):
        b, row, col = addr(c, masked)
        return ref.at[b, pl.ds(row, _CR), pl.ds(col, _CC)]

    def m_in(c, slot):
        return pltpu.make_async_copy(hbm_ref(x_hbm, c, 1), bufs[slot], isems[slot])

    def m_out(c, slot):
        return pltpu.make_async_copy(bufs[slot], hbm_ref(o_hbm, c, 1), osems[slot])

    def c_in(c, slot):
        return pltpu.make_async_copy(
            hbm_ref(x_hbm, c, 0), spm.at[sid, slot], cisems[slot]
        )

    def c_out(c, slot):
        return pltpu.make_async_copy(
            spm.at[sid, slot], hbm_ref(o_hbm, c, 0), cosems[slot]
        )

    def process(h, s, drain, prefetch):
        slot_p = (s + _PREF) % _NB
        if drain:
            m_out(h - (_NB - _PREF), slot_p).wait()
            c_out(h - (_NB - _PREF), slot_p).wait()
        if prefetch:
            m_in(h + _PREF, slot_p).start()
            c_in(h + _PREF, slot_p).start()
        c_in(h, s).wait()
        c_out(h, s).start()
        m_in(h, s).wait()
        _mask_chunk(bufs[s])
        m_out(h, s).start()

    for s in range(_PREF):
        m_in(s, s).start()
        c_in(s, s).start()

    for s in range(_NB):
        process(s, s, drain=(s >= _NB - _PREF), prefetch=True)

    def step(k, _):
        for s in range(_NB):
            process(k * _NB + s, s, drain=True, prefetch=True)
        return _

    lax.fori_loop(1, _NCH // _NB - 1, step, 0)

    last = _NCH - _NB
    for s in range(_NB):
        h = last + s
        process(h, s, drain=(h + _PREF < _NCH), prefetch=(h + _PREF < _NCH))

    for s in range(_NB):
        h = last + s
        m_out(h, s).wait()
        c_out(h, s).wait()


_sc_kernel = functools.partial(
    pl.kernel,
    out_type=jax.ShapeDtypeStruct((_B, _S, _D), jnp.float32),
    mesh=plsc.VectorSubcoreMesh(core_axis_name="c", subcore_axis_name="s"),
    scratch_types=(
        [pltpu.VMEM_SHARED((_NS, _NB, _CR, _CC), jnp.float32)]
        + [pltpu.VMEM((_CR, _CC), jnp.float32)] * _NB
        + [pltpu.SemaphoreType.DMA] * (4 * _NB)
    ),
    compiler_params=pltpu.CompilerParams(use_tc_tiling_on_sc=True),
)(_sc_body)


def kernel(x):
    return _sc_kernel(x)


# speedup vs baseline: 1.0546x; 1.0022x over previous
import functools

import jax
import jax.numpy as jnp
from jax import lax
from jax.experimental import pallas as pl
from jax.experimental.pallas import tpu as pltpu
from jax.experimental.pallas import tpu_sc as plsc

_THRESHOLD = 0.1

_B = 4
_S = 4096
_D = 4096
_HALF_ROWS = _S // 2
_NW = 32
_BAND = _HALF_ROWS // _NW
_CR = 8
_CC = 1024
_RC = _BAND // _CR
_CCN = _D // _CC
_NCH = _B * _RC * _CCN
_NB = 8
_PREF = 6

_TC_BLK = 512


def _mask_chunk(buf):
    @plsc.parallel_loop(0, _CR * _CC, 16, unroll=8)
    def _m(i):
        r = i // _CC
        c = pl.multiple_of(i % _CC, 16)
        v = buf[r, pl.ds(c, 16)]
        buf[r, pl.ds(c, 16)] = jnp.where(jnp.abs(v) > _THRESHOLD, v, 0.0)


def _sc_body(x_hbm, o_hbm, *scratch):
    bufs = scratch[:_NB]
    isems = scratch[_NB:2 * _NB]
    osems = scratch[2 * _NB:3 * _NB]
    wid = lax.axis_index("s") * 2 + lax.axis_index("c")

    def hbm_ref(ref, c):
        b = c // (_RC * _CCN)
        rc = (c // _CCN) % _RC
        cc = c % _CCN
        row = _HALF_ROWS + wid * _BAND + rc * _CR
        return ref.at[b, pl.ds(pl.multiple_of(row, _CR), _CR),
                      pl.ds(cc * _CC, _CC)]

    def m_in(c, slot):
        return pltpu.make_async_copy(hbm_ref(x_hbm, c), bufs[slot], isems[slot])

    def m_out(c, slot):
        return pltpu.make_async_copy(bufs[slot], hbm_ref(o_hbm, c), osems[slot])

    def process(h, s, drain, prefetch):
        slot_p = (s + _PREF) % _NB
        if drain:
            m_out(h - (_NB - _PREF), slot_p).wait()
        if prefetch:
            m_in(h + _PREF, slot_p).start()
        m_in(h, s).wait()
        _mask_chunk(bufs[s])
        m_out(h, s).start()

    for s in range(_PREF):
        m_in(s, s).start()

    for s in range(_NB):
        process(s, s, drain=(s >= _NB - _PREF), prefetch=True)

    def step(k, _):
        for s in range(_NB):
            process(k * _NB + s, s, drain=True, prefetch=True)
        return _

    lax.fori_loop(1, _NCH // _NB - 1, step, 0)

    last = _NCH - _NB
    for s in range(_NB):
        h = last + s
        process(h, s, drain=(h + _PREF < _NCH), prefetch=(h + _PREF < _NCH))

    for s in range(_NB):
        m_out(last + s, s).wait()


_sc_mask = functools.partial(
    pl.kernel,
    out_type=jax.ShapeDtypeStruct((_B, _S, _D), jnp.float32),
    mesh=plsc.VectorSubcoreMesh(core_axis_name="c", subcore_axis_name="s"),
    scratch_types=(
        [pltpu.VMEM((_CR, _CC), jnp.float32)] * _NB
        + [pltpu.SemaphoreType.DMA] * (2 * _NB)
    ),
    compiler_params=pltpu.CompilerParams(use_tc_tiling_on_sc=True),
)(_sc_body)


def _tc_copy_body(x_ref, y_ref, o_ref):
    del y_ref
    o_ref[...] = x_ref[...]


def _tc_copy(x, y):
    return pl.pallas_call(
        _tc_copy_body,
        grid=(_B, _HALF_ROWS // _TC_BLK),
        in_specs=[
            pl.BlockSpec((1, _TC_BLK, _D), lambda i, j: (i, j, 0)),
            pl.BlockSpec(memory_space=pl.ANY),
        ],
        out_specs=pl.BlockSpec((1, _TC_BLK, _D), lambda i, j: (i, j, 0)),
        out_shape=jax.ShapeDtypeStruct((_B, _S, _D), jnp.float32),
        input_output_aliases={1: 0},
    )(x, y)


def kernel(x):
    return _tc_copy(x, _sc_mask(x))
